# baseline (speedup 1.0000x reference)
import functools

import jax
import jax.numpy as jnp
from jax import lax
from jax.experimental import pallas as pl
from jax.experimental.pallas import tpu as pltpu
from jax.experimental.pallas import tpu_sc as plsc


def _make_gather(B, D, num_workers, chunk):
    b_per_w = B // num_workers
    nstep = b_per_w // chunk
    mesh = plsc.VectorSubcoreMesh(core_axis_name="c", subcore_axis_name="s")

    @functools.partial(
        pl.kernel,
        mesh=mesh,
        compiler_params=pltpu.CompilerParams(use_tc_tiling_on_sc=False),
        out_type=jax.ShapeDtypeStruct((B, D), jnp.float32),
        scratch_types=[
            pltpu.VMEM((chunk,), jnp.int32),
            pltpu.VMEM((chunk, D), jnp.float32),
            pltpu.SemaphoreType.DMA,
        ],
    )
    def gather(idx_hbm, table_hbm, out_hbm, idx_v, rows_v, sem):
        wid = lax.axis_index("s") * 2 + lax.axis_index("c")
        base = wid * b_per_w
        for s in range(nstep):
            off = base + s * chunk
            pltpu.sync_copy(idx_hbm.at[pl.ds(off, chunk)], idx_v)
            pltpu.async_copy(table_hbm.at[idx_v], rows_v, sem).wait()
            pltpu.sync_copy(rows_v, out_hbm.at[pl.ds(off, chunk)])

    return gather


def kernel(x, table):
    B = x.size
    D = table.shape[1]
    idx = x.reshape(B).astype(jnp.int32)
    out = _make_gather(B, D, 32, 2560)(idx, table)
    return out.reshape(*x.shape, D)


# speedup vs baseline: 1.0142x; 1.0142x over previous
import functools

import jax
import jax.numpy as jnp
from jax import lax
from jax.experimental import pallas as pl
from jax.experimental.pallas import tpu as pltpu
from jax.experimental.pallas import tpu_sc as plsc


def _make_gather(B, D, num_workers, chunk, nbuf):
    b_per_w = B // num_workers
    nstep = b_per_w // chunk
    mesh = plsc.VectorSubcoreMesh(core_axis_name="c", subcore_axis_name="s")

    @functools.partial(
        pl.kernel,
        mesh=mesh,
        compiler_params=pltpu.CompilerParams(use_tc_tiling_on_sc=False),
        out_type=jax.ShapeDtypeStruct((B, D), jnp.float32),
        scratch_types=[
            pltpu.VMEM((b_per_w,), jnp.int32),
            pltpu.VMEM((nbuf, chunk, D), jnp.float32),
            pltpu.SemaphoreType.DMA((nbuf,)),
            pltpu.SemaphoreType.DMA((nbuf,)),
        ],
    )
    def gather(idx_hbm, table_hbm, out_hbm, idx_v, rows_v, gsem, wsem):
        wid = lax.axis_index("s") * 2 + lax.axis_index("c")
        base = wid * b_per_w
        pltpu.sync_copy(idx_hbm.at[pl.ds(base, b_per_w)], idx_v)

        def start_gather(g):
            b = g % nbuf
            return pltpu.async_copy(
                table_hbm.at[idx_v.at[pl.ds(g * chunk, chunk)]],
                rows_v.at[b],
                gsem.at[b],
            )

        def start_write(g):
            b = g % nbuf
            return pltpu.async_copy(
                rows_v.at[b],
                out_hbm.at[pl.ds(base + g * chunk, chunk)],
                wsem.at[b],
            )

        gh = [None] * nstep
        wh = [None] * nstep
        for g in range(min(nbuf, nstep)):
            gh[g] = start_gather(g)
        for g in range(nstep):
            gh[g].wait()
            wh[g] = start_write(g)
            nxt = g + nbuf
            if nxt < nstep:
                wh[g].wait()
                gh[nxt] = start_gather(nxt)
        for g in range(max(0, nstep - nbuf), nstep):
            wh[g].wait()

    return gather


def kernel(x, table):
    B = x.size
    D = table.shape[1]
    idx = x.reshape(B).astype(jnp.int32)
    out = _make_gather(B, D, 32, 1280, 4)(idx, table)
    return out.reshape(*x.shape, D)
